# Initial kernel scaffold; baseline (speedup 1.0000x reference)
import jax
import jax.numpy as jnp
from jax.experimental import pallas as pl


def kernel(x, edge_index0, edge_index1, W1, b1, W2, b2):
    raise NotImplementedError("write your pallas kernel here")


# speedup vs baseline: 3.9051x; 3.9051x over previous
import functools

import jax
import jax.numpy as jnp
from jax import lax
from jax.experimental import pallas as pl
from jax.experimental.pallas import tpu as pltpu
from jax.experimental.pallas import tpu_sc as plsc

N = 10000
E = 320000
D_IN = 128
D_HID = 128
D_OUT = 64

NC = 2
NS = 16
NW = NC * NS
CHUNK = 128
EPW = E // NW
CPW = -(-EPW // CHUNK)
NPAD = NW * CHUNK * CPW // NW
NPAD = 10112
DUMMY = N
ROWS_PT = NPAD // NS
RB = NPAD // 8


def _mesh():
    return plsc.VectorSubcoreMesh(core_axis_name="c", subcore_axis_name="s")


@functools.lru_cache(maxsize=None)
def _hist_call():
    def body(idx_hbm, zeros_hbm, onehot_hbm, out_hbm,
             hist, idx_all, vals):
        c = lax.axis_index("c")
        s = lax.axis_index("s")
        wid = c * NS + s
        pltpu.sync_copy(zeros_hbm, hist.at[pl.ds(s * ROWS_PT, ROWS_PT)])
        plsc.subcore_barrier()
        for a in range(4):
            pltpu.sync_copy(onehot_hbm.at[a], vals)
            pltpu.sync_copy(idx_hbm.at[a, wid], idx_all)

            def step(k, carry):
                pltpu.sync_copy(vals, hist.at[idx_all.at[k]], add=True)
                return carry

            lax.fori_loop(0, CPW, step, 0)
        plsc.subcore_barrier()
        pltpu.sync_copy(hist.at[pl.ds(s * ROWS_PT, ROWS_PT)],
                        out_hbm.at[c, pl.ds(s * ROWS_PT, ROWS_PT)])

    return pl.kernel(
        body,
        out_type=jax.ShapeDtypeStruct((NC, NPAD, 128), jnp.float32),
        mesh=_mesh(),
        scratch_types=[
            pltpu.VMEM_SHARED((NPAD, 128), jnp.float32),
            pltpu.VMEM((CPW, CHUNK), jnp.int32),
            pltpu.VMEM((CHUNK, 128), jnp.float32),
        ],
    )


@functools.lru_cache(maxsize=None)
def _agg_call(d):
    def body(feat_hbm, src_hbm, dst_hbm, zeros_hbm, out_hbm,
             acc, sidx_all, didx_all, rows, sem):
        c = lax.axis_index("c")
        s = lax.axis_index("s")
        wid = c * NS + s
        pltpu.sync_copy(zeros_hbm,
                        acc.at[pl.ds(s * ROWS_PT, ROWS_PT)])
        pltpu.sync_copy(src_hbm.at[wid], sidx_all)
        pltpu.sync_copy(dst_hbm.at[wid], didx_all)
        plsc.subcore_barrier()

        def step(k, carry):
            pltpu.async_copy(feat_hbm.at[sidx_all.at[k]], rows, sem).wait()
            pltpu.sync_copy(rows, acc.at[didx_all.at[k]], add=True)
            return carry

        lax.fori_loop(0, CPW, step, 0)
        plsc.subcore_barrier()
        pltpu.sync_copy(acc.at[pl.ds(s * ROWS_PT, ROWS_PT)],
                        out_hbm.at[c, pl.ds(s * ROWS_PT, ROWS_PT)])

    return pl.kernel(
        body,
        out_type=jax.ShapeDtypeStruct((NC, NPAD, d), jnp.float32),
        mesh=_mesh(),
        scratch_types=[
            pltpu.VMEM_SHARED((NPAD, d), jnp.float32),
            pltpu.VMEM((CPW, CHUNK), jnp.int32),
            pltpu.VMEM((CPW, CHUNK), jnp.int32),
            pltpu.VMEM((CHUNK, d), jnp.float32),
            pltpu.SemaphoreType.DMA,
        ],
    )


def _norm(da, db, a):
    c = 32 * a
    return lax.rsqrt(jnp.maximum(da[:, c:c + 1] + db[:, c:c + 1], 1.0))


def _tc1_body(x_ref, da_ref, db_ref, o_ref):
    o_ref[...] = x_ref[...] * _norm(da_ref, db_ref, 0)


def _tc2_body(p0, p1, da, db, w1, b1, o):
    agg = p0[...] + p1[...]
    z = jnp.dot(agg, w1[...], preferred_element_type=jnp.float32)
    t = jnp.maximum(z * _norm(da, db, 1) + b1[...], 0.0)
    o[...] = t * _norm(da, db, 2)


def _tc3_body(q0, q1, da, db, w2, b2, o):
    agg = q0[...] + q1[...]
    z = jnp.dot(agg, w2[...], preferred_element_type=jnp.float32)
    o[...] = z * _norm(da, db, 3) + b2[...]


def _rows_spec(cols):
    return pl.BlockSpec((RB, cols), lambda i: (i, 0))


def _full_spec(r, c):
    return pl.BlockSpec((r, c), lambda i: (0, 0))


@functools.lru_cache(maxsize=None)
def _tc1_call():
    return pl.pallas_call(
        _tc1_body,
        grid=(NPAD // RB,),
        in_specs=[_rows_spec(D_IN), _rows_spec(128), _rows_spec(128)],
        out_specs=_rows_spec(D_IN),
        out_shape=jax.ShapeDtypeStruct((NPAD, D_IN), jnp.float32),
    )


@functools.lru_cache(maxsize=None)
def _tc2_call():
    return pl.pallas_call(
        _tc2_body,
        grid=(NPAD // RB,),
        in_specs=[_rows_spec(D_HID), _rows_spec(D_HID),
                  _rows_spec(128), _rows_spec(128),
                  _full_spec(D_IN, D_HID), _full_spec(1, D_HID)],
        out_specs=_rows_spec(D_HID),
        out_shape=jax.ShapeDtypeStruct((NPAD, D_HID), jnp.float32),
    )


@functools.lru_cache(maxsize=None)
def _tc3_call():
    return pl.pallas_call(
        _tc3_body,
        grid=(NPAD // RB,),
        in_specs=[_rows_spec(D_HID), _rows_spec(D_HID),
                  _rows_spec(128), _rows_spec(128),
                  _full_spec(D_HID, D_OUT), _full_spec(1, D_OUT)],
        out_specs=_rows_spec(D_OUT),
        out_shape=jax.ShapeDtypeStruct((NPAD, D_OUT), jnp.float32),
    )


def _prep_edges(ei):
    pad = ((0, 0), (0, CPW * CHUNK - EPW))
    src = jnp.pad(ei[0].reshape(NW, EPW), pad,
                  constant_values=DUMMY).reshape(NW, CPW, CHUNK)
    dst = jnp.pad(ei[1].reshape(NW, EPW), pad,
                  constant_values=DUMMY).reshape(NW, CPW, CHUNK)
    return src, dst


def kernel(x, edge_index0, edge_index1, W1, b1, W2, b2):
    s0, d0 = _prep_edges(edge_index0)
    s1, d1 = _prep_edges(edge_index1)
    idx4 = jnp.stack([s0, d0, s1, d1])
    zeros128 = jnp.zeros((ROWS_PT, D_HID), jnp.float32)
    onehot4 = jnp.zeros((4, CHUNK, 128), jnp.float32)
    for a in range(4):
        onehot4 = onehot4.at[a, :, 32 * a].set(1.0)
    xpad = jnp.pad(x, ((0, NPAD - N), (0, 0)))

    degp = _hist_call()(idx4, zeros128, onehot4)

    h0 = _tc1_call()(xpad, degp[0], degp[1])
    p = _agg_call(D_HID)(h0, s0, d0, zeros128)
    t = _tc2_call()(p[0], p[1], degp[0], degp[1],
                    W1, b1.reshape(1, D_HID))
    q = _agg_call(D_HID)(t, s1, d1, zeros128)
    out = _tc3_call()(q[0], q[1], degp[0], degp[1],
                      W2, b2.reshape(1, D_OUT))
    return out[:N]
